# baseline (speedup 1.0000x reference)
import functools

import jax
import jax.numpy as jnp
from jax import lax
from jax.experimental import pallas as pl
from jax.experimental.pallas import tpu as pltpu
from jax.experimental.pallas import tpu_sc as plsc

B, N, D_AUDIO, D_TEXT, VOCAB = 4, 8, 768, 512, 49408
BN = B * N
V_BLK = 6176
N_BLK = VOCAB // V_BLK


def _score_body(kw_ref, w_ref, b_ref, tab_ref, idx_ref, qn_s, max_s, arg_s):
    pid = pl.program_id(0)

    @pl.when(pid == 0)
    def _prologue():
        proj = jnp.dot(kw_ref[...], w_ref[...],
                       precision=lax.Precision.HIGHEST,
                       preferred_element_type=jnp.float32) + b_ref[...][None, :]
        nrm = jnp.sqrt(jnp.sum(proj * proj, axis=1, keepdims=True))
        qn_s[...] = proj / jnp.maximum(nrm, 1e-8)
        max_s[...] = jnp.full((BN,), -jnp.inf, dtype=jnp.float32)
        arg_s[...] = jnp.full((BN,), VOCAB, dtype=jnp.int32)

    block = tab_ref[...]
    norm2 = jnp.sum(block * block, axis=1)
    inv = 1.0 / jnp.maximum(jnp.sqrt(norm2), 1e-8)
    scores = lax.dot_general(qn_s[...], block,
                             (((1,), (1,)), ((), ())),
                             precision=lax.Precision.HIGHEST,
                             preferred_element_type=jnp.float32)
    scores = scores * inv[None, :]
    bmax = jnp.max(scores, axis=1)
    barg = jnp.argmax(scores, axis=1).astype(jnp.int32)
    cand = pid * V_BLK + barg
    improved = (bmax > max_s[...]) | ((bmax == max_s[...]) & (cand < arg_s[...]))
    max_s[...] = jnp.where(improved, bmax, max_s[...])
    arg_s[...] = jnp.where(improved, cand, arg_s[...])

    @pl.when(pid == N_BLK - 1)
    def _epilogue():
        idx_ref[...] = arg_s[...]


def _tc_scores(kw, w, b, table):
    return pl.pallas_call(
        _score_body,
        grid=(N_BLK,),
        in_specs=[
            pl.BlockSpec((BN, D_AUDIO), lambda i: (0, 0)),
            pl.BlockSpec((D_AUDIO, D_TEXT), lambda i: (0, 0)),
            pl.BlockSpec((D_TEXT,), lambda i: (0,)),
            pl.BlockSpec((V_BLK, D_TEXT), lambda i: (i, 0)),
        ],
        out_specs=pl.BlockSpec((BN,), lambda i: (0,)),
        out_shape=jax.ShapeDtypeStruct((BN,), jnp.int32),
        scratch_shapes=[
            pltpu.VMEM((BN, D_TEXT), jnp.float32),
            pltpu.VMEM((BN,), jnp.float32),
            pltpu.VMEM((BN,), jnp.int32),
        ],
        compiler_params=pltpu.CompilerParams(
            dimension_semantics=("arbitrary",),
        ),
    )(kw, w, b, table)


ROWS_PER_W = 8
N_WORKERS = BN // ROWS_PER_W


def _sc_body(tab_hbm, idx_hbm, out_hbm, idx_v, rows_v, sem):
    wid = lax.axis_index("s") * 2 + lax.axis_index("c")

    @pl.when(wid < N_WORKERS)
    def _work():
        base = wid * ROWS_PER_W
        pltpu.sync_copy(idx_hbm.at[pl.ds(base, ROWS_PER_W)], idx_v)
        pltpu.async_copy(tab_hbm.at[idx_v], rows_v, sem).wait()
        pltpu.sync_copy(rows_v, out_hbm.at[pl.ds(base, ROWS_PER_W)])


@functools.cache
def _sc_gather():
    return pl.kernel(
        _sc_body,
        out_type=jax.ShapeDtypeStruct((BN, D_TEXT), jnp.float32),
        mesh=plsc.VectorSubcoreMesh(core_axis_name="c", subcore_axis_name="s"),
        scratch_types=[
            pltpu.VMEM((ROWS_PER_W,), jnp.int32),
            pltpu.VMEM((ROWS_PER_W, D_TEXT), jnp.float32),
            pltpu.SemaphoreType.DMA,
        ],
    )


def kernel(keywords, W, b, token_embedding):
    kw = keywords.reshape(BN, D_AUDIO)
    idx = _tc_scores(kw, W, b, token_embedding)
    out = _sc_gather()(token_embedding, idx)
    return out.reshape(B, N, D_TEXT)


# speedup vs baseline: 1.5385x; 1.5385x over previous
import functools

import jax
import jax.numpy as jnp
from jax import lax
from jax.experimental import pallas as pl
from jax.experimental.pallas import tpu as pltpu
from jax.experimental.pallas import tpu_sc as plsc

B, N, D_AUDIO, D_TEXT, VOCAB = 4, 8, 768, 512, 49408
BN = B * N
V_BLK = 6176
N_BLK = VOCAB // V_BLK


def _score_body(kw_ref, w_ref, b_ref, tab_ref, idx_ref, qn_s, max_s, arg_s):
    pid = pl.program_id(0)

    @pl.when(pid == 0)
    def _prologue():
        proj = jnp.dot(kw_ref[...], w_ref[...],
                       precision=lax.Precision.HIGHEST,
                       preferred_element_type=jnp.float32) + b_ref[...][None, :]
        nrm = jnp.sqrt(jnp.sum(proj * proj, axis=1, keepdims=True))
        qn_s[...] = proj / jnp.maximum(nrm, 1e-8)
        max_s[...] = jnp.full((BN,), -jnp.inf, dtype=jnp.float32)
        arg_s[...] = jnp.full((BN,), VOCAB, dtype=jnp.int32)

    block = tab_ref[...]
    norm2 = jnp.sum(block * block, axis=1)
    inv = 1.0 / jnp.maximum(jnp.sqrt(norm2), 1e-8)
    qn = qn_s[...]
    qh = qn.astype(jnp.bfloat16)
    ql = (qn - qh.astype(jnp.float32)).astype(jnp.bfloat16)
    bh = block.astype(jnp.bfloat16)
    bl = (block - bh.astype(jnp.float32)).astype(jnp.bfloat16)
    dn = (((1,), (1,)), ((), ()))
    scores = (lax.dot_general(qh, bl, dn, preferred_element_type=jnp.float32)
              + lax.dot_general(ql, bh, dn, preferred_element_type=jnp.float32))
    scores = scores + lax.dot_general(qh, bh, dn,
                                      preferred_element_type=jnp.float32)
    scores = scores * inv[None, :]
    bmax = jnp.max(scores, axis=1)
    barg = jnp.argmax(scores, axis=1).astype(jnp.int32)
    cand = pid * V_BLK + barg
    improved = (bmax > max_s[...]) | ((bmax == max_s[...]) & (cand < arg_s[...]))
    max_s[...] = jnp.where(improved, bmax, max_s[...])
    arg_s[...] = jnp.where(improved, cand, arg_s[...])

    @pl.when(pid == N_BLK - 1)
    def _epilogue():
        idx_ref[...] = arg_s[...]


def _tc_scores(kw, w, b, table):
    return pl.pallas_call(
        _score_body,
        grid=(N_BLK,),
        in_specs=[
            pl.BlockSpec((BN, D_AUDIO), lambda i: (0, 0)),
            pl.BlockSpec((D_AUDIO, D_TEXT), lambda i: (0, 0)),
            pl.BlockSpec((D_TEXT,), lambda i: (0,)),
            pl.BlockSpec((V_BLK, D_TEXT), lambda i: (i, 0)),
        ],
        out_specs=pl.BlockSpec((BN,), lambda i: (0,)),
        out_shape=jax.ShapeDtypeStruct((BN,), jnp.int32),
        scratch_shapes=[
            pltpu.VMEM((BN, D_TEXT), jnp.float32),
            pltpu.VMEM((BN,), jnp.float32),
            pltpu.VMEM((BN,), jnp.int32),
        ],
        compiler_params=pltpu.CompilerParams(
            dimension_semantics=("arbitrary",),
        ),
    )(kw, w, b, table)


ROWS_PER_W = 8
N_WORKERS = BN // ROWS_PER_W


def _sc_body(tab_hbm, idx_hbm, out_hbm, idx_v, rows_v, sem):
    wid = lax.axis_index("s") * 2 + lax.axis_index("c")

    @pl.when(wid < N_WORKERS)
    def _work():
        base = wid * ROWS_PER_W
        pltpu.sync_copy(idx_hbm.at[pl.ds(base, ROWS_PER_W)], idx_v)
        pltpu.async_copy(tab_hbm.at[idx_v], rows_v, sem).wait()
        pltpu.sync_copy(rows_v, out_hbm.at[pl.ds(base, ROWS_PER_W)])


@functools.cache
def _sc_gather():
    return pl.kernel(
        _sc_body,
        out_type=jax.ShapeDtypeStruct((BN, D_TEXT), jnp.float32),
        mesh=plsc.VectorSubcoreMesh(core_axis_name="c", subcore_axis_name="s"),
        scratch_types=[
            pltpu.VMEM((ROWS_PER_W,), jnp.int32),
            pltpu.VMEM((ROWS_PER_W, D_TEXT), jnp.float32),
            pltpu.SemaphoreType.DMA,
        ],
    )


def kernel(keywords, W, b, token_embedding):
    kw = keywords.reshape(BN, D_AUDIO)
    idx = _tc_scores(kw, W, b, token_embedding)
    out = _sc_gather()(token_embedding, idx)
    return out.reshape(B, N, D_TEXT)


# speedup vs baseline: 1.9198x; 1.2478x over previous
import functools

import jax
import jax.numpy as jnp
from jax import lax
from jax.experimental import pallas as pl
from jax.experimental.pallas import tpu as pltpu
from jax.experimental.pallas import tpu_sc as plsc

B, N, D_AUDIO, D_TEXT, VOCAB = 4, 8, 768, 512, 49408
BN = B * N
V_BLK = 6176
N_BLK = VOCAB // V_BLK
NEG = float("-inf")


def _score_body(kw_ref, w_ref, b_ref, tab_ref, idx2_ref, qn_ref,
                qn_s, m1_s, i1_s, m2_s, i2_s):
    pid = pl.program_id(0)

    @pl.when(pid == 0)
    def _prologue():
        proj = jnp.dot(kw_ref[...], w_ref[...],
                       precision=lax.Precision.HIGHEST,
                       preferred_element_type=jnp.float32) + b_ref[...][None, :]
        nrm = jnp.sqrt(jnp.sum(proj * proj, axis=1, keepdims=True))
        qn = proj / jnp.maximum(nrm, 1e-8)
        qn_s[...] = qn
        qn_ref[...] = qn
        m1_s[...] = jnp.full((BN,), NEG, dtype=jnp.float32)
        m2_s[...] = jnp.full((BN,), NEG, dtype=jnp.float32)
        i1_s[...] = jnp.full((BN,), VOCAB, dtype=jnp.int32)
        i2_s[...] = jnp.full((BN,), VOCAB, dtype=jnp.int32)

    block = tab_ref[...]
    norm2 = jnp.sum(block * block, axis=1)
    inv = 1.0 / jnp.maximum(jnp.sqrt(norm2), 1e-8)
    scores = lax.dot_general(qn_s[...], block,
                             (((1,), (1,)), ((), ())),
                             preferred_element_type=jnp.float32)
    scores = scores * inv[None, :]

    b1 = jnp.max(scores, axis=1)
    a1 = jnp.argmax(scores, axis=1).astype(jnp.int32)
    lane = lax.broadcasted_iota(jnp.int32, (BN, V_BLK), 1)
    scores2 = jnp.where(lane == a1[:, None], NEG, scores)
    b2 = jnp.max(scores2, axis=1)
    a2 = jnp.argmax(scores2, axis=1).astype(jnp.int32)
    j1 = pid * V_BLK + a1
    j2 = pid * V_BLK + a2

    m1, i1, m2, i2 = m1_s[...], i1_s[...], m2_s[...], i2_s[...]
    gt = (b1 > m1) | ((b1 == m1) & (j1 < i1))
    w1 = jnp.where(gt, b1, m1)
    wi1 = jnp.where(gt, j1, i1)
    l1 = jnp.where(gt, m1, b1)
    li1 = jnp.where(gt, i1, j1)
    gt2 = (b2 > m2) | ((b2 == m2) & (j2 < i2))
    w2 = jnp.where(gt2, b2, m2)
    wi2 = jnp.where(gt2, j2, i2)
    gt3 = (w2 > l1) | ((w2 == l1) & (wi2 < li1))
    m1_s[...] = w1
    i1_s[...] = wi1
    m2_s[...] = jnp.where(gt3, w2, l1)
    i2_s[...] = jnp.where(gt3, wi2, li1)

    @pl.when(pid == N_BLK - 1)
    def _epilogue():
        ia, ib = i1_s[...], i2_s[...]
        idx2_ref[0, :] = jnp.minimum(ia, ib)
        idx2_ref[1, :] = jnp.maximum(ia, ib)


def _tc_scores(kw, w, b, table):
    return pl.pallas_call(
        _score_body,
        grid=(N_BLK,),
        in_specs=[
            pl.BlockSpec((BN, D_AUDIO), lambda i: (0, 0)),
            pl.BlockSpec((D_AUDIO, D_TEXT), lambda i: (0, 0)),
            pl.BlockSpec((D_TEXT,), lambda i: (0,)),
            pl.BlockSpec((V_BLK, D_TEXT), lambda i: (i, 0)),
        ],
        out_specs=[
            pl.BlockSpec((2, BN), lambda i: (0, 0)),
            pl.BlockSpec((BN, D_TEXT), lambda i: (0, 0)),
        ],
        out_shape=[
            jax.ShapeDtypeStruct((2, BN), jnp.int32),
            jax.ShapeDtypeStruct((BN, D_TEXT), jnp.float32),
        ],
        scratch_shapes=[
            pltpu.VMEM((BN, D_TEXT), jnp.float32),
            pltpu.VMEM((BN,), jnp.float32),
            pltpu.VMEM((BN,), jnp.int32),
            pltpu.VMEM((BN,), jnp.float32),
            pltpu.VMEM((BN,), jnp.int32),
        ],
        compiler_params=pltpu.CompilerParams(
            dimension_semantics=("arbitrary",),
        ),
    )(kw, w, b, table)


ROWS_PER_W = 8
N_WORKERS = BN // ROWS_PER_W
LN = 16


def _lane_sum(x):
    lane = lax.iota(jnp.int32, LN)
    for k in (8, 4, 2, 1):
        perm = (lane + k) & (LN - 1)
        x = x + x.at[perm].get(mode="promise_in_bounds")
    return x


def _sc_body(tab_hbm, idx2_hbm, qn_hbm, out_hbm,
             ilo_v, ihi_v, rlo_v, rhi_v, qn_v, out_v, sem):
    wid = lax.axis_index("s") * 2 + lax.axis_index("c")

    @pl.when(wid < N_WORKERS)
    def _work():
        base = wid * ROWS_PER_W
        pltpu.sync_copy(idx2_hbm.at[0, pl.ds(base, ROWS_PER_W)], ilo_v)
        pltpu.sync_copy(idx2_hbm.at[1, pl.ds(base, ROWS_PER_W)], ihi_v)
        c_lo = pltpu.async_copy(tab_hbm.at[ilo_v], rlo_v, sem)
        c_hi = pltpu.async_copy(tab_hbm.at[ihi_v], rhi_v, sem)
        pltpu.sync_copy(qn_hbm.at[pl.ds(base, ROWS_PER_W)], qn_v)
        c_lo.wait()
        c_hi.wait()
        for r in range(ROWS_PER_W):
            s0 = jnp.zeros((LN,), jnp.float32)
            s1 = jnp.zeros((LN,), jnp.float32)
            n0 = jnp.zeros((LN,), jnp.float32)
            n1 = jnp.zeros((LN,), jnp.float32)
            for c in range(0, D_TEXT, LN):
                q = qn_v[r, pl.ds(c, LN)]
                e0 = rlo_v[r, pl.ds(c, LN)]
                e1 = rhi_v[r, pl.ds(c, LN)]
                s0 = s0 + q * e0
                s1 = s1 + q * e1
                n0 = n0 + e0 * e0
                n1 = n1 + e1 * e1
            d0 = _lane_sum(s0)
            d1 = _lane_sum(s1)
            q0 = _lane_sum(n0)
            q1 = _lane_sum(n1)
            g0 = d0 * jnp.abs(d0) * q1
            g1 = d1 * jnp.abs(d1) * q0
            take_hi = g1 > g0
            for c in range(0, D_TEXT, LN):
                out_v[r, pl.ds(c, LN)] = jnp.where(
                    take_hi, rhi_v[r, pl.ds(c, LN)], rlo_v[r, pl.ds(c, LN)])
        pltpu.sync_copy(out_v, out_hbm.at[pl.ds(base, ROWS_PER_W)])


@functools.cache
def _sc_rescore():
    return pl.kernel(
        _sc_body,
        out_type=jax.ShapeDtypeStruct((BN, D_TEXT), jnp.float32),
        mesh=plsc.VectorSubcoreMesh(core_axis_name="c", subcore_axis_name="s"),
        scratch_types=[
            pltpu.VMEM((ROWS_PER_W,), jnp.int32),
            pltpu.VMEM((ROWS_PER_W,), jnp.int32),
            pltpu.VMEM((ROWS_PER_W, D_TEXT), jnp.float32),
            pltpu.VMEM((ROWS_PER_W, D_TEXT), jnp.float32),
            pltpu.VMEM((ROWS_PER_W, D_TEXT), jnp.float32),
            pltpu.VMEM((ROWS_PER_W, D_TEXT), jnp.float32),
            pltpu.SemaphoreType.DMA,
        ],
    )


def kernel(keywords, W, b, token_embedding):
    kw = keywords.reshape(BN, D_AUDIO)
    idx2, qn = _tc_scores(kw, W, b, token_embedding)
    out = _sc_rescore()(token_embedding, idx2, qn)
    return out.reshape(B, N, D_TEXT)
